# Initial kernel scaffold; baseline (speedup 1.0000x reference)
import jax
import jax.numpy as jnp
from jax.experimental import pallas as pl


def kernel(x, edge_index, batch, W1, b1, W2, b2, Wout, bout):
    raise NotImplementedError("write your pallas kernel here")


# speedup vs baseline: 11.0786x; 11.0786x over previous
import functools

import jax
import jax.numpy as jnp
from jax import lax
from jax.experimental import pallas as pl
from jax.experimental.pallas import tpu as pltpu
from jax.experimental.pallas import tpu_sc as plsc

N = 10000
D = 128
G = 64
NP = 10112
E = 320000
EP = 323584
NSC = 2
NT = 16
EPT = EP // (NSC * NT)
CH = 128
NCH = EPT // CH
RPT = NP // NT

_mesh = plsc.VectorSubcoreMesh(core_axis_name="c", subcore_axis_name="s")


@functools.partial(
    pl.kernel,
    out_type=jax.ShapeDtypeStruct((NSC, NP, D), jnp.float32),
    mesh=_mesh,
    scratch_types=[
        pltpu.VMEM((CH,), jnp.int32),
        pltpu.VMEM((CH, D), jnp.float32),
        pltpu.VMEM_SHARED((NP, D), jnp.float32),
        pltpu.SemaphoreType.DMA,
    ],
)
def _deg_kernel(dst_hbm, ones_hbm, zeros_hbm, out_hbm, dst_v, ones_v, acc, sem):
    c = lax.axis_index("c")
    s = lax.axis_index("s")
    rbase = s * RPT

    @pl.when(c == 0)
    def _():
        pltpu.sync_copy(ones_hbm.at[pl.ds(rbase, RPT)], acc.at[pl.ds(rbase, RPT)])

    @pl.when(c != 0)
    def _():
        pltpu.sync_copy(zeros_hbm.at[pl.ds(rbase, RPT)], acc.at[pl.ds(rbase, RPT)])

    pltpu.sync_copy(ones_hbm.at[pl.ds(0, CH)], ones_v)
    plsc.subcore_barrier()

    ebase = (c * NT + s) * EPT

    def body(i, _):
        off = ebase + i * CH
        pltpu.sync_copy(dst_hbm.at[pl.ds(off, CH)], dst_v)
        pltpu.sync_copy(ones_v, acc.at[dst_v], add=True)
        return 0

    lax.fori_loop(0, NCH, body, 0)
    plsc.subcore_barrier()
    pltpu.sync_copy(acc.at[pl.ds(rbase, RPT)], out_hbm.at[c].at[pl.ds(rbase, RPT)])


@functools.partial(
    pl.kernel,
    out_type=jax.ShapeDtypeStruct((NSC, NP, D), jnp.float32),
    mesh=_mesh,
    scratch_types=[
        pltpu.VMEM((CH,), jnp.int32),
        pltpu.VMEM((CH,), jnp.int32),
        pltpu.VMEM((CH, D), jnp.float32),
        pltpu.VMEM_SHARED((NP, D), jnp.float32),
        pltpu.SemaphoreType.DMA,
    ],
)
def _msg_kernel(hp_hbm, src_hbm, dst_hbm, zeros_hbm, out_hbm,
                src_v, dst_v, rows, acc, sem):
    c = lax.axis_index("c")
    s = lax.axis_index("s")
    rbase = s * RPT
    @pl.when(c == 0)
    def _():
        pltpu.sync_copy(hp_hbm.at[pl.ds(rbase, RPT)], acc.at[pl.ds(rbase, RPT)])

    @pl.when(c != 0)
    def _():
        pltpu.sync_copy(zeros_hbm.at[pl.ds(rbase, RPT)], acc.at[pl.ds(rbase, RPT)])

    plsc.subcore_barrier()

    ebase = (c * NT + s) * EPT

    def body(i, _):
        off = ebase + i * CH
        pltpu.sync_copy(src_hbm.at[pl.ds(off, CH)], src_v)
        pltpu.sync_copy(dst_hbm.at[pl.ds(off, CH)], dst_v)
        pltpu.async_copy(hp_hbm.at[src_v], rows, sem).wait()
        pltpu.sync_copy(rows, acc.at[dst_v], add=True)
        return 0

    lax.fori_loop(0, NCH, body, 0)
    plsc.subcore_barrier()
    pltpu.sync_copy(acc.at[pl.ds(rbase, RPT)], out_hbm.at[c].at[pl.ds(rbase, RPT)])


def _dis_col(degp_ref):
    deg = degp_ref[0, :, 0:1] + degp_ref[1, :, 0:1]
    return lax.rsqrt(deg)


def _mm_scale_body(x_ref, w_ref, degp_ref, o_ref):
    dis = _dis_col(degp_ref)
    h = jnp.dot(x_ref[...], w_ref[...], preferred_element_type=jnp.float32)
    o_ref[...] = h * dis


def _layer2_body(p_ref, degp_ref, b_ref, w_ref, o_ref):
    dis = _dis_col(degp_ref)
    g = jax.nn.relu((p_ref[0] + p_ref[1]) * dis + b_ref[...])
    o_ref[...] = jnp.dot(g, w_ref[...], preferred_element_type=jnp.float32) * dis


def _final_body(p_ref, degp_ref, b_ref, batch_ref, wout_ref, bout_ref, o_ref):
    dis = _dis_col(degp_ref)
    g = jax.nn.relu((p_ref[0] + p_ref[1]) * dis + b_ref[...])
    ids = lax.broadcasted_iota(jnp.int32, (G, NP), 0)
    sel = (ids == batch_ref[...]).astype(jnp.float32)
    counts = jnp.sum(sel, axis=1, keepdims=True)
    pooled = jnp.dot(sel, g, preferred_element_type=jnp.float32)
    pooled = pooled / jnp.maximum(counts, 1.0)
    o_ref[...] = jnp.dot(pooled, wout_ref[...],
                         preferred_element_type=jnp.float32) + bout_ref[...]


_mm_scale = pl.pallas_call(
    _mm_scale_body, out_shape=jax.ShapeDtypeStruct((NP, D), jnp.float32))
_layer2 = pl.pallas_call(
    _layer2_body, out_shape=jax.ShapeDtypeStruct((NP, D), jnp.float32))
_final = pl.pallas_call(
    _final_body, out_shape=jax.ShapeDtypeStruct((G, D), jnp.float32))


def kernel(x, edge_index, batch, W1, b1, W2, b2, Wout, bout):
    f32 = jnp.float32
    src = edge_index[0]
    dst = edge_index[1]
    epad = jnp.full((EP - E,), N, dtype=jnp.int32)
    src_p = jnp.concatenate([src, epad])
    dst_p = jnp.concatenate([dst, epad])
    x_p = jnp.concatenate([x, jnp.zeros((NP - N, D), f32)])
    batch2d = jnp.concatenate(
        [batch, jnp.full((NP - N,), G, dtype=jnp.int32)]).reshape(1, NP)
    zeros128 = jnp.zeros((NP, D), f32)
    ones128 = jnp.ones((NP, D), f32)

    degp = _deg_kernel(dst_p, ones128, zeros128)
    hp1 = _mm_scale(x_p, W1, degp)
    p1 = _msg_kernel(hp1, src_p, dst_p, zeros128)
    hp2 = _layer2(p1, degp, b1.reshape(1, D), W2)
    p2 = _msg_kernel(hp2, src_p, dst_p, zeros128)
    return _final(p2, degp, b2.reshape(1, D), batch2d, Wout,
                  bout.reshape(1, D))


# speedup vs baseline: 13.3650x; 1.2064x over previous
import functools

import jax
import jax.numpy as jnp
from jax import lax
from jax.experimental import pallas as pl
from jax.experimental.pallas import tpu as pltpu
from jax.experimental.pallas import tpu_sc as plsc

N = 10000
D = 128
G = 64
NP = 10112
E = 320000
EP = 323584
NSC = 2
NT = 16
EPT = EP // (NSC * NT)
CH = 128
NCH = EPT // CH
RPT = NP // NT

_mesh = plsc.VectorSubcoreMesh(core_axis_name="c", subcore_axis_name="s")


@functools.partial(
    pl.kernel,
    out_type=jax.ShapeDtypeStruct((NSC, NP, D), jnp.float32),
    mesh=_mesh,
    scratch_types=[
        pltpu.VMEM((NCH, CH), jnp.int32),
        pltpu.VMEM((CH, D), jnp.float32),
        pltpu.VMEM_SHARED((NP, D), jnp.float32),
        pltpu.SemaphoreType.DMA,
    ],
)
def _deg_kernel(dst_hbm, ones_hbm, zeros_hbm, out_hbm, dst_i, ones_v, acc, sem):
    c = lax.axis_index("c")
    s = lax.axis_index("s")
    wid = c * NT + s
    rbase = s * RPT

    @pl.when(c == 0)
    def _():
        pltpu.sync_copy(ones_hbm.at[pl.ds(rbase, RPT)], acc.at[pl.ds(rbase, RPT)])

    @pl.when(c != 0)
    def _():
        pltpu.sync_copy(zeros_hbm.at[pl.ds(rbase, RPT)], acc.at[pl.ds(rbase, RPT)])

    pltpu.sync_copy(ones_hbm.at[pl.ds(0, CH)], ones_v)
    pltpu.sync_copy(dst_hbm.at[wid], dst_i)
    plsc.subcore_barrier()

    def body(i, _):
        pltpu.sync_copy(ones_v, acc.at[dst_i.at[i]], add=True)
        return 0

    lax.fori_loop(0, NCH, body, 0)
    plsc.subcore_barrier()
    pltpu.sync_copy(acc.at[pl.ds(rbase, RPT)], out_hbm.at[c].at[pl.ds(rbase, RPT)])


@functools.partial(
    pl.kernel,
    out_type=jax.ShapeDtypeStruct((NSC, NP, D), jnp.float32),
    mesh=_mesh,
    scratch_types=[
        pltpu.VMEM((CH,), jnp.int32),
        pltpu.VMEM((CH,), jnp.int32),
        pltpu.VMEM((NCH, CH), jnp.int32),
        pltpu.VMEM((CH, D), jnp.float32),
        pltpu.VMEM((CH, D), jnp.float32),
        pltpu.VMEM_SHARED((NP, D), jnp.float32),
        pltpu.SemaphoreType.DMA,
        pltpu.SemaphoreType.DMA,
    ],
)
def _msg_kernel(hp_hbm, src_hbm, dst_hbm, zeros_hbm, out_hbm,
                src_a, src_b, dst_i, rows_a, rows_b, acc, sem_a, sem_b):
    c = lax.axis_index("c")
    s = lax.axis_index("s")
    wid = c * NT + s
    rbase = s * RPT
    @pl.when(c == 0)
    def _():
        pltpu.sync_copy(hp_hbm.at[pl.ds(rbase, RPT)], acc.at[pl.ds(rbase, RPT)])

    @pl.when(c != 0)
    def _():
        pltpu.sync_copy(zeros_hbm.at[pl.ds(rbase, RPT)], acc.at[pl.ds(rbase, RPT)])

    pltpu.sync_copy(dst_hbm.at[wid], dst_i)
    plsc.subcore_barrier()

    pltpu.sync_copy(src_hbm.at[wid].at[0], src_a)
    pltpu.async_copy(hp_hbm.at[src_a], rows_a, sem_a)

    def body(k, _):
        j = 2 * k
        pltpu.sync_copy(src_hbm.at[wid].at[j + 1], src_b)
        pltpu.async_copy(hp_hbm.at[src_b], rows_b, sem_b)
        pltpu.make_async_copy(hp_hbm.at[src_a], rows_a, sem_a).wait()
        pltpu.sync_copy(rows_a, acc.at[dst_i.at[j]], add=True)
        pltpu.sync_copy(src_hbm.at[wid].at[j + 2], src_a)
        pltpu.async_copy(hp_hbm.at[src_a], rows_a, sem_a)
        pltpu.make_async_copy(hp_hbm.at[src_b], rows_b, sem_b).wait()
        pltpu.sync_copy(rows_b, acc.at[dst_i.at[j + 1]], add=True)
        return 0

    lax.fori_loop(0, (NCH - 1) // 2, body, 0)
    pltpu.make_async_copy(hp_hbm.at[src_a], rows_a, sem_a).wait()
    pltpu.sync_copy(rows_a, acc.at[dst_i.at[NCH - 1]], add=True)
    plsc.subcore_barrier()
    pltpu.sync_copy(acc.at[pl.ds(rbase, RPT)], out_hbm.at[c].at[pl.ds(rbase, RPT)])


def _dis_col(degp_ref):
    deg = degp_ref[0, :, 0:1] + degp_ref[1, :, 0:1]
    return lax.rsqrt(deg)


def _mm_scale_body(x_ref, w_ref, degp_ref, o_ref):
    dis = _dis_col(degp_ref)
    h = jnp.dot(x_ref[...], w_ref[...], preferred_element_type=jnp.float32)
    o_ref[...] = h * dis


def _layer2_body(p_ref, degp_ref, b_ref, w_ref, o_ref):
    dis = _dis_col(degp_ref)
    g = jax.nn.relu((p_ref[0] + p_ref[1]) * dis + b_ref[...])
    o_ref[...] = jnp.dot(g, w_ref[...], preferred_element_type=jnp.float32) * dis


def _final_body(p_ref, degp_ref, b_ref, batch_ref, wout_ref, bout_ref, o_ref):
    dis = _dis_col(degp_ref)
    g = jax.nn.relu((p_ref[0] + p_ref[1]) * dis + b_ref[...])
    ids = lax.broadcasted_iota(jnp.int32, (G, NP), 0)
    sel = (ids == batch_ref[...]).astype(jnp.float32)
    counts = jnp.sum(sel, axis=1, keepdims=True)
    pooled = jnp.dot(sel, g, preferred_element_type=jnp.float32)
    pooled = pooled / jnp.maximum(counts, 1.0)
    o_ref[...] = jnp.dot(pooled, wout_ref[...],
                         preferred_element_type=jnp.float32) + bout_ref[...]


_mm_scale = pl.pallas_call(
    _mm_scale_body, out_shape=jax.ShapeDtypeStruct((NP, D), jnp.float32))
_layer2 = pl.pallas_call(
    _layer2_body, out_shape=jax.ShapeDtypeStruct((NP, D), jnp.float32))
_final = pl.pallas_call(
    _final_body, out_shape=jax.ShapeDtypeStruct((G, D), jnp.float32))


def kernel(x, edge_index, batch, W1, b1, W2, b2, Wout, bout):
    f32 = jnp.float32
    src = edge_index[0]
    dst = edge_index[1]
    epad = jnp.full((EP - E,), N, dtype=jnp.int32)
    src_p = jnp.concatenate([src, epad]).reshape(NSC * NT, NCH, CH)
    dst_p = jnp.concatenate([dst, epad]).reshape(NSC * NT, NCH, CH)
    x_p = jnp.concatenate([x, jnp.zeros((NP - N, D), f32)])
    batch2d = jnp.concatenate(
        [batch, jnp.full((NP - N,), G, dtype=jnp.int32)]).reshape(1, NP)
    zeros128 = jnp.zeros((NP, D), f32)
    ones128 = jnp.ones((NP, D), f32)

    degp = _deg_kernel(dst_p, ones128, zeros128)
    hp1 = _mm_scale(x_p, W1, degp)
    p1 = _msg_kernel(hp1, src_p, dst_p, zeros128)
    hp2 = _layer2(p1, degp, b1.reshape(1, D), W2)
    p2 = _msg_kernel(hp2, src_p, dst_p, zeros128)
    return _final(p2, degp, b2.reshape(1, D), batch2d, Wout,
                  bout.reshape(1, D))


# speedup vs baseline: 13.6917x; 1.0244x over previous
import functools

import jax
import jax.numpy as jnp
from jax import lax
from jax.experimental import pallas as pl
from jax.experimental.pallas import tpu as pltpu
from jax.experimental.pallas import tpu_sc as plsc

N = 10000
D = 128
G = 64
NP = 10112
E = 320000
EP = 323584
NSC = 2
NT = 16
EPT = EP // (NSC * NT)
CH = 128
NCH = EPT // CH
RPT = NP // NT

_mesh = plsc.VectorSubcoreMesh(core_axis_name="c", subcore_axis_name="s")


@functools.partial(
    pl.kernel,
    out_type=jax.ShapeDtypeStruct((NSC, NP, D), jnp.float32),
    mesh=_mesh,
    scratch_types=[
        pltpu.VMEM((NCH, CH), jnp.int32),
        pltpu.VMEM((CH, D), jnp.float32),
        pltpu.VMEM_SHARED((NP, D), jnp.float32),
        pltpu.SemaphoreType.DMA,
    ],
)
def _deg_kernel(dst_hbm, ones_hbm, zeros_hbm, out_hbm, dst_i, ones_v, acc, sem):
    c = lax.axis_index("c")
    s = lax.axis_index("s")
    wid = c * NT + s
    rbase = s * RPT

    @pl.when(c == 0)
    def _():
        pltpu.sync_copy(ones_hbm.at[pl.ds(rbase, RPT)], acc.at[pl.ds(rbase, RPT)])

    @pl.when(c != 0)
    def _():
        pltpu.sync_copy(zeros_hbm.at[pl.ds(rbase, RPT)], acc.at[pl.ds(rbase, RPT)])

    pltpu.sync_copy(ones_hbm.at[pl.ds(0, CH)], ones_v)
    pltpu.sync_copy(dst_hbm.at[wid], dst_i)
    plsc.subcore_barrier()

    def body(i, _):
        pltpu.sync_copy(ones_v, acc.at[dst_i.at[i]], add=True)
        return 0

    lax.fori_loop(0, NCH, body, 0)
    plsc.subcore_barrier()
    pltpu.sync_copy(acc.at[pl.ds(rbase, RPT)], out_hbm.at[c].at[pl.ds(rbase, RPT)])


@functools.partial(
    pl.kernel,
    out_type=jax.ShapeDtypeStruct((NSC, NP, D), jnp.float32),
    mesh=_mesh,
    scratch_types=[
        pltpu.VMEM((CH,), jnp.int32),
        pltpu.VMEM((CH,), jnp.int32),
        pltpu.VMEM((NCH, CH), jnp.int32),
        pltpu.VMEM((CH, D), jnp.float32),
        pltpu.VMEM((CH, D), jnp.float32),
        pltpu.VMEM_SHARED((NP, D), jnp.float32),
        pltpu.SemaphoreType.DMA,
        pltpu.SemaphoreType.DMA,
    ],
)
def _msg_kernel(hp_hbm, src_hbm, dst_hbm, zeros_hbm, out_hbm,
                src_a, src_b, dst_i, rows_a, rows_b, acc, sem_a, sem_b):
    c = lax.axis_index("c")
    s = lax.axis_index("s")
    wid = c * NT + s
    rbase = s * RPT
    pltpu.sync_copy(zeros_hbm.at[pl.ds(rbase, RPT)], acc.at[pl.ds(rbase, RPT)])
    pltpu.sync_copy(dst_hbm.at[wid], dst_i)
    plsc.subcore_barrier()

    pltpu.sync_copy(src_hbm.at[wid].at[0], src_a)
    pltpu.async_copy(hp_hbm.at[src_a], rows_a, sem_a)

    def body(k, _):
        j = 2 * k
        pltpu.sync_copy(src_hbm.at[wid].at[j + 1], src_b)
        pltpu.async_copy(hp_hbm.at[src_b], rows_b, sem_b)
        pltpu.make_async_copy(hp_hbm.at[src_a], rows_a, sem_a).wait()
        pltpu.sync_copy(rows_a, acc.at[dst_i.at[j]], add=True)
        pltpu.sync_copy(src_hbm.at[wid].at[j + 2], src_a)
        pltpu.async_copy(hp_hbm.at[src_a], rows_a, sem_a)
        pltpu.make_async_copy(hp_hbm.at[src_b], rows_b, sem_b).wait()
        pltpu.sync_copy(rows_b, acc.at[dst_i.at[j + 1]], add=True)
        return 0

    lax.fori_loop(0, (NCH - 1) // 2, body, 0)
    pltpu.make_async_copy(hp_hbm.at[src_a], rows_a, sem_a).wait()
    pltpu.sync_copy(rows_a, acc.at[dst_i.at[NCH - 1]], add=True)
    plsc.subcore_barrier()
    pltpu.sync_copy(acc.at[pl.ds(rbase, RPT)], out_hbm.at[c].at[pl.ds(rbase, RPT)])


def _dis_col(degp_ref):
    deg = degp_ref[0, :, 0:1] + degp_ref[1, :, 0:1]
    return lax.rsqrt(deg)


def _mm_scale_body(x_ref, w_ref, degp_ref, o_ref):
    dis = _dis_col(degp_ref)
    h = jnp.dot(x_ref[...], w_ref[...], preferred_element_type=jnp.float32)
    o_ref[...] = h * dis


def _layer2_body(p_ref, hp_ref, degp_ref, b_ref, w_ref, o_ref):
    dis = _dis_col(degp_ref)
    g = jax.nn.relu((p_ref[0] + p_ref[1] + hp_ref[...]) * dis + b_ref[...])
    o_ref[...] = jnp.dot(g, w_ref[...], preferred_element_type=jnp.float32) * dis


def _final_body(p_ref, hp_ref, degp_ref, b_ref, batch_ref, wout_ref, bout_ref, o_ref):
    dis = _dis_col(degp_ref)
    g = jax.nn.relu((p_ref[0] + p_ref[1] + hp_ref[...]) * dis + b_ref[...])
    ids = lax.broadcasted_iota(jnp.int32, (G, NP), 0)
    sel = (ids == batch_ref[...]).astype(jnp.float32)
    counts = jnp.sum(sel, axis=1, keepdims=True)
    pooled = jnp.dot(sel, g, preferred_element_type=jnp.float32)
    pooled = pooled / jnp.maximum(counts, 1.0)
    o_ref[...] = jnp.dot(pooled, wout_ref[...],
                         preferred_element_type=jnp.float32) + bout_ref[...]


_mm_scale = pl.pallas_call(
    _mm_scale_body, out_shape=jax.ShapeDtypeStruct((NP, D), jnp.float32))
_layer2 = pl.pallas_call(
    _layer2_body, out_shape=jax.ShapeDtypeStruct((NP, D), jnp.float32))
_final = pl.pallas_call(
    _final_body, out_shape=jax.ShapeDtypeStruct((G, D), jnp.float32))


def kernel(x, edge_index, batch, W1, b1, W2, b2, Wout, bout):
    f32 = jnp.float32
    src = edge_index[0]
    dst = edge_index[1]
    epad = jnp.full((EP - E,), N, dtype=jnp.int32)
    src_p = jnp.concatenate([src, epad]).reshape(NSC * NT, NCH, CH)
    dst_p = jnp.concatenate([dst, epad]).reshape(NSC * NT, NCH, CH)
    x_p = jnp.concatenate([x, jnp.zeros((NP - N, D), f32)])
    batch2d = jnp.concatenate(
        [batch, jnp.full((NP - N,), G, dtype=jnp.int32)]).reshape(1, NP)
    zeros128 = jnp.zeros((NP, D), f32)
    ones128 = jnp.ones((NP, D), f32)

    degp = _deg_kernel(dst_p, ones128, zeros128)
    hp1 = _mm_scale(x_p, W1, degp)
    p1 = _msg_kernel(hp1, src_p, dst_p, zeros128)
    hp2 = _layer2(p1, hp1, degp, b1.reshape(1, D), W2)
    p2 = _msg_kernel(hp2, src_p, dst_p, zeros128)
    return _final(p2, hp2, degp, b2.reshape(1, D), batch2d, Wout,
                  bout.reshape(1, D))
